# Initial kernel scaffold; baseline (speedup 1.0000x reference)
import jax
import jax.numpy as jnp
from jax.experimental import pallas as pl


def kernel(dynamic_indices, static_indices, time, data_emb_table, static_emb_table, time_w, time_b):
    raise NotImplementedError("write your pallas kernel here")


# speedup vs baseline: 2.7485x; 2.7485x over previous
import functools

import jax
import jax.numpy as jnp
from jax import lax
from jax.experimental import pallas as pl
from jax.experimental.pallas import tpu as pltpu
from jax.experimental.pallas import tpu_sc as plsc

B, S, M, H = 1024, 200, 4, 128
N_STATIC = 8
NC, NS = 2, 16
NW = NC * NS
ROWS_PER_W = B // NW
E = 40
G = 20
NG = E // G
CHUNKS = S // E
HC = H // 16


def _sc_body(dyn_hbm, sidx_hbm, t16_hbm, tbl_hbm, stbl_hbm, tw_hbm, tb_hbm,
             out_hbm, idx_v, rows_v, out_v, t16_v, sidx_v, srows_v, wtb_v, sem):
    wid = lax.axis_index("s") * NC + lax.axis_index("c")

    pltpu.sync_copy(tw_hbm, wtb_v.at[0])
    pltpu.sync_copy(tb_hbm, wtb_v.at[1])
    wch = [wtb_v[0, pl.ds(h * 16, 16)] for h in range(HC)]
    tbch = [wtb_v[1, pl.ds(h * 16, 16)] for h in range(HC)]

    def row_body(r, _):
        b = wid * ROWS_PER_W + r
        pltpu.sync_copy(sidx_hbm.at[pl.ds(b * N_STATIC, N_STATIC)], sidx_v)
        pltpu.async_copy(stbl_hbm.at[sidx_v], srows_v, sem).wait()

        base = []
        for h in range(HC):
            hs = pl.ds(h * 16, 16)
            acc = srows_v[0, hs]
            for j in range(1, N_STATIC):
                acc = acc + srows_v[j, hs]
            base.append(acc * (1.0 / N_STATIC) + tbch[h])

        def chunk_body(ci, _):
            e0 = b * S + ci * E
            cps = []
            for j in range(NG):
                pltpu.sync_copy(
                    dyn_hbm.at[pl.ds((e0 + j * G) * M, G * M)], idx_v.at[j])
                cps.append(pltpu.async_copy(
                    tbl_hbm.at[idx_v.at[j]],
                    rows_v.at[pl.ds(j * G * M, G * M)], sem))
            pltpu.sync_copy(t16_hbm.at[pl.ds(e0 * 16, E * 16)], t16_v)
            for cp in cps:
                cp.wait()

            def ev_body(e, _):
                t = t16_v[pl.ds(e * 16, 16)]
                for h in range(HC):
                    hs = pl.ds(h * 16, 16)
                    v01 = rows_v[e * M + 0, hs] + rows_v[e * M + 1, hs]
                    v23 = rows_v[e * M + 2, hs] + rows_v[e * M + 3, hs]
                    out_v[e, hs] = v01 + v23 + base[h] + t * wch[h]
                return 0

            lax.fori_loop(0, E, ev_body, 0)
            pltpu.sync_copy(out_v, out_hbm.at[pl.ds(e0, E)])
            return 0

        lax.fori_loop(0, CHUNKS, chunk_body, 0)
        return 0

    lax.fori_loop(0, ROWS_PER_W, row_body, 0)


@jax.jit
def _run(dyn_idx, static_idx, time16, data_tbl, static_tbl, time_w, time_b):
    mesh = plsc.VectorSubcoreMesh(core_axis_name="c", subcore_axis_name="s")
    kfn = functools.partial(
        pl.kernel,
        mesh=mesh,
        out_type=jax.ShapeDtypeStruct((B * S, H), jnp.float32),
        scratch_types=[
            pltpu.VMEM((NG, G * M), jnp.int32),
            pltpu.VMEM((E * M, H), jnp.float32),
            pltpu.VMEM((E, H), jnp.float32),
            pltpu.VMEM((E * 16,), jnp.float32),
            pltpu.VMEM((N_STATIC,), jnp.int32),
            pltpu.VMEM((N_STATIC, H), jnp.float32),
            pltpu.VMEM((2, H), jnp.float32),
            pltpu.SemaphoreType.DMA,
        ],
    )(_sc_body)
    return kfn(dyn_idx, static_idx, time16, data_tbl, static_tbl, time_w, time_b)


def kernel(dynamic_indices, static_indices, time, data_emb_table,
           static_emb_table, time_w, time_b):
    time16 = jnp.broadcast_to(time.reshape(-1)[:, None], (B * S, 16)).reshape(-1)
    out = _run(
        dynamic_indices.reshape(-1),
        static_indices.reshape(-1),
        time16,
        data_emb_table,
        static_emb_table,
        time_w.reshape(-1),
        time_b.reshape(-1),
    )
    return out.reshape(B, S, H)


# speedup vs baseline: 3.6650x; 1.3335x over previous
import functools

import jax
import jax.numpy as jnp
from jax import lax
from jax.experimental import pallas as pl
from jax.experimental.pallas import tpu as pltpu
from jax.experimental.pallas import tpu_sc as plsc

B, S, M, H = 1024, 200, 4, 128
N_STATIC = 8
NC, NS = 2, 16
NW = NC * NS
ROWS_PER_W = B // NW
E = 40
G = 20
NG = E // G
CHUNKS = S // E
HC = H // 16


def _sc_body(dyn_hbm, sidx_hbm, t16_hbm, tbl_hbm, stbl_hbm, tw_hbm, tb_hbm,
             out_hbm, ridx_v, rows_v, out_v, t16_v, sidx_v, srows_v, wtb_v,
             gsem0, gsem1, ssem, osem0, osem1):
    wid = lax.axis_index("s") * NC + lax.axis_index("c")
    gsem = (gsem0, gsem1)
    osem = (osem0, osem1)

    pltpu.sync_copy(tw_hbm, wtb_v.at[0])
    pltpu.sync_copy(tb_hbm, wtb_v.at[1])
    wch = [wtb_v[0, pl.ds(h * 16, 16)] for h in range(HC)]
    tbch = [wtb_v[1, pl.ds(h * 16, 16)] for h in range(HC)]

    def start_gathers(ci, p):
        cps = []
        for j in range(NG):
            cps.append(pltpu.async_copy(
                tbl_hbm.at[ridx_v.at[pl.ds((ci * E + j * G) * M, G * M)]],
                rows_v.at[p, pl.ds(j * G * M, G * M)], gsem[p]))
        return cps

    def row_body(r, _):
        b = wid * ROWS_PER_W + r
        pltpu.sync_copy(sidx_hbm.at[pl.ds(b * N_STATIC, N_STATIC)], sidx_v)
        scp = pltpu.async_copy(stbl_hbm.at[sidx_v], srows_v, ssem)
        pltpu.sync_copy(dyn_hbm.at[pl.ds(b * S * M, S * M)], ridx_v)
        g_pend = {0: start_gathers(0, 0)}
        pltpu.sync_copy(t16_hbm.at[pl.ds(b * S * 16, S * 16)], t16_v)
        scp.wait()

        base = []
        for h in range(HC):
            hs = pl.ds(h * 16, 16)
            acc = srows_v[0, hs]
            for j in range(1, N_STATIC):
                acc = acc + srows_v[j, hs]
            base.append(acc * (1.0 / N_STATIC) + tbch[h])

        out_pend = [None, None]
        for ci in range(CHUNKS):
            p = ci % 2
            if ci + 1 < CHUNKS:
                g_pend[ci + 1] = start_gathers(ci + 1, 1 - p)
            for cp in g_pend.pop(ci):
                cp.wait()
            if out_pend[p] is not None:
                out_pend[p].wait()
                out_pend[p] = None

            def ev_body(e2, _, ci=ci, p=p):
                for sub in range(2):
                    e = e2 * 2 + sub
                    t = t16_v[pl.ds((ci * E + e) * 16, 16)]
                    for h in range(HC):
                        hs = pl.ds(h * 16, 16)
                        v01 = rows_v[p, e * M + 0, hs] + rows_v[p, e * M + 1, hs]
                        v23 = rows_v[p, e * M + 2, hs] + rows_v[p, e * M + 3, hs]
                        out_v[p, e, hs] = v01 + v23 + base[h] + t * wch[h]
                return 0

            lax.fori_loop(0, E // 2, ev_body, 0)
            out_pend[p] = pltpu.async_copy(
                out_v.at[p], out_hbm.at[pl.ds(b * S + ci * E, E)], osem[p])

        for p in range(2):
            if out_pend[p] is not None:
                out_pend[p].wait()
        return 0

    lax.fori_loop(0, ROWS_PER_W, row_body, 0)


@jax.jit
def _run(dyn_idx, static_idx, time16, data_tbl, static_tbl, time_w, time_b):
    mesh = plsc.VectorSubcoreMesh(core_axis_name="c", subcore_axis_name="s")
    kfn = functools.partial(
        pl.kernel,
        mesh=mesh,
        out_type=jax.ShapeDtypeStruct((B * S, H), jnp.float32),
        scratch_types=[
            pltpu.VMEM((S * M,), jnp.int32),
            pltpu.VMEM((2, E * M, H), jnp.float32),
            pltpu.VMEM((2, E, H), jnp.float32),
            pltpu.VMEM((S * 16,), jnp.float32),
            pltpu.VMEM((N_STATIC,), jnp.int32),
            pltpu.VMEM((N_STATIC, H), jnp.float32),
            pltpu.VMEM((2, H), jnp.float32),
            pltpu.SemaphoreType.DMA,
            pltpu.SemaphoreType.DMA,
            pltpu.SemaphoreType.DMA,
            pltpu.SemaphoreType.DMA,
            pltpu.SemaphoreType.DMA,
        ],
    )(_sc_body)
    return kfn(dyn_idx, static_idx, time16, data_tbl, static_tbl, time_w, time_b)


def kernel(dynamic_indices, static_indices, time, data_emb_table,
           static_emb_table, time_w, time_b):
    time16 = jnp.broadcast_to(time.reshape(-1)[:, None], (B * S, 16)).reshape(-1)
    out = _run(
        dynamic_indices.reshape(-1),
        static_indices.reshape(-1),
        time16,
        data_emb_table,
        static_emb_table,
        time_w.reshape(-1),
        time_b.reshape(-1),
    )
    return out.reshape(B, S, H)


# speedup vs baseline: 5.5677x; 1.5192x over previous
import functools

import jax
import jax.numpy as jnp
from jax import lax
from jax.experimental import pallas as pl
from jax.experimental.pallas import tpu as pltpu
from jax.experimental.pallas import tpu_sc as plsc

B, S, M, H = 1024, 200, 4, 128
N_STATIC = 8
NC, NS = 2, 16
NW = NC * NS
ROWS_PER_W = B // NW
E = 40
G = 20
NG = E // G
CHUNKS = S // E
HC = H // 16


def _sc_body(dyn_hbm, sidx_hbm, t16_hbm, tbl_hbm, stbl_hbm, tw_hbm, tb_hbm,
             out_hbm, ridx_v, rows_v, out_v, t16_v, sidx_v, srows_v, wtb_v,
             gsem0, gsem1, ssem, osem0, osem1):
    wid = lax.axis_index("s") * NC + lax.axis_index("c")
    gsem = (gsem0, gsem1)
    osem = (osem0, osem1)

    pltpu.sync_copy(tw_hbm, wtb_v.at[0])
    pltpu.sync_copy(tb_hbm, wtb_v.at[1])
    wch = [wtb_v[0, pl.ds(h * 16, 16)] for h in range(HC)]
    tbch = [wtb_v[1, pl.ds(h * 16, 16)] for h in range(HC)]

    def start_gathers(ci, p):
        cps = []
        for j in range(NG):
            cps.append(pltpu.async_copy(
                tbl_hbm.at[ridx_v.at[pl.ds((ci * E + j * G) * M, G * M)]],
                rows_v.at[p, pl.ds(j * G * M, G * M)], gsem[p]))
        return cps

    def row_body(r, _):
        b = wid * ROWS_PER_W + r
        pltpu.sync_copy(sidx_hbm.at[pl.ds(b * N_STATIC, N_STATIC)], sidx_v)
        scp = pltpu.async_copy(stbl_hbm.at[sidx_v], srows_v, ssem)
        pltpu.sync_copy(dyn_hbm.at[pl.ds(b * S * M, S * M)], ridx_v)
        g_pend = {0: start_gathers(0, 0)}
        pltpu.sync_copy(t16_hbm.at[pl.ds(b * S * 16, S * 16)], t16_v)
        scp.wait()

        base = []
        for h in range(HC):
            hs = pl.ds(h * 16, 16)
            acc = srows_v[0, hs]
            for j in range(1, N_STATIC):
                acc = acc + srows_v[j, hs]
            base.append(acc * (1.0 / N_STATIC) + tbch[h])

        out_pend = [None, None]
        for ci in range(CHUNKS):
            p = ci % 2
            if ci + 1 < CHUNKS:
                g_pend[ci + 1] = start_gathers(ci + 1, 1 - p)
            for cp in g_pend.pop(ci):
                cp.wait()
            if out_pend[p] is not None:
                out_pend[p].wait()
                out_pend[p] = None

            @plsc.parallel_loop(0, E, unroll=2)
            def ev_body(e, ci=ci, p=p):
                t = t16_v[pl.ds((ci * E + e) * 16, 16)]
                for h in range(HC):
                    hs = pl.ds(h * 16, 16)
                    v01 = rows_v[p, e * M + 0, hs] + rows_v[p, e * M + 1, hs]
                    v23 = rows_v[p, e * M + 2, hs] + rows_v[p, e * M + 3, hs]
                    out_v[p, e, hs] = v01 + v23 + base[h] + t * wch[h]
            out_pend[p] = pltpu.async_copy(
                out_v.at[p], out_hbm.at[pl.ds(b * S + ci * E, E)], osem[p])

        for p in range(2):
            if out_pend[p] is not None:
                out_pend[p].wait()
        return 0

    lax.fori_loop(0, ROWS_PER_W, row_body, 0)


@jax.jit
def _run(dyn_idx, static_idx, time16, data_tbl, static_tbl, time_w, time_b):
    mesh = plsc.VectorSubcoreMesh(core_axis_name="c", subcore_axis_name="s")
    kfn = functools.partial(
        pl.kernel,
        mesh=mesh,
        out_type=jax.ShapeDtypeStruct((B * S, H), jnp.float32),
        scratch_types=[
            pltpu.VMEM((S * M,), jnp.int32),
            pltpu.VMEM((2, E * M, H), jnp.float32),
            pltpu.VMEM((2, E, H), jnp.float32),
            pltpu.VMEM((S * 16,), jnp.float32),
            pltpu.VMEM((N_STATIC,), jnp.int32),
            pltpu.VMEM((N_STATIC, H), jnp.float32),
            pltpu.VMEM((2, H), jnp.float32),
            pltpu.SemaphoreType.DMA,
            pltpu.SemaphoreType.DMA,
            pltpu.SemaphoreType.DMA,
            pltpu.SemaphoreType.DMA,
            pltpu.SemaphoreType.DMA,
        ],
    )(_sc_body)
    return kfn(dyn_idx, static_idx, time16, data_tbl, static_tbl, time_w, time_b)


def kernel(dynamic_indices, static_indices, time, data_emb_table,
           static_emb_table, time_w, time_b):
    time16 = jnp.broadcast_to(time.reshape(-1)[:, None], (B * S, 16)).reshape(-1)
    out = _run(
        dynamic_indices.reshape(-1),
        static_indices.reshape(-1),
        time16,
        data_emb_table,
        static_emb_table,
        time_w.reshape(-1),
        time_b.reshape(-1),
    )
    return out.reshape(B, S, H)


# speedup vs baseline: 7.2544x; 1.3029x over previous
import functools

import jax
import jax.numpy as jnp
from jax import lax
from jax.experimental import pallas as pl
from jax.experimental.pallas import tpu as pltpu
from jax.experimental.pallas import tpu_sc as plsc

B, S, M, H = 1024, 200, 4, 128
N_STATIC = 8
NC, NS = 2, 16
NW = NC * NS
ROWS_PER_W = B // NW
E = 40
G = 20
NG = E // G
CHUNKS = S // E
HC = H // 16


def _sc_body(dyn_hbm, sidx_hbm, t_hbm, tbl_hbm, stbl_hbm, tw_hbm, tb_hbm,
             out_hbm, ridx_v, rows_v, out_v, t_v, sidx_v, srows_v, wtb_v,
             gsem0, gsem1, ssem, osem0, osem1):
    wid = lax.axis_index("s") * NC + lax.axis_index("c")
    gsem = (gsem0, gsem1)
    osem = (osem0, osem1)

    pltpu.sync_copy(tw_hbm.at[0], wtb_v.at[0])
    pltpu.sync_copy(tb_hbm, wtb_v.at[1])
    wch = [wtb_v[0, pl.ds(h * 16, 16)] for h in range(HC)]
    tbch = [wtb_v[1, pl.ds(h * 16, 16)] for h in range(HC)]
    lanes = lax.iota(jnp.int32, 16)

    def start_gathers(ci, p):
        cps = []
        for j in range(NG):
            cps.append(pltpu.async_copy(
                tbl_hbm.at[ridx_v.at[pl.ds((ci * E + j * G) * M, G * M)]],
                rows_v.at[p, pl.ds(j * G * M, G * M)], gsem[p]))
        return cps

    def row_body(r, _):
        b = wid * ROWS_PER_W + r
        pltpu.sync_copy(sidx_hbm.at[b], sidx_v)
        scp = pltpu.async_copy(stbl_hbm.at[sidx_v], srows_v, ssem)
        pltpu.sync_copy(dyn_hbm.at[pl.ds(b * S * M, S * M)], ridx_v)
        g_pend = {0: start_gathers(0, 0)}
        pltpu.sync_copy(t_hbm.at[pl.ds(b * S, S)], t_v.at[pl.ds(0, S)])
        scp.wait()

        base = []
        for h in range(HC):
            hs = pl.ds(h * 16, 16)
            acc = srows_v[0, hs]
            for j in range(1, N_STATIC):
                acc = acc + srows_v[j, hs]
            base.append(acc * (1.0 / N_STATIC) + tbch[h])

        out_pend = [None, None]
        for ci in range(CHUNKS):
            p = ci % 2
            if ci + 1 < CHUNKS:
                g_pend[ci + 1] = start_gathers(ci + 1, 1 - p)
            for cp in g_pend.pop(ci):
                cp.wait()
            if out_pend[p] is not None:
                out_pend[p].wait()
                out_pend[p] = None

            @plsc.parallel_loop(0, E, unroll=2)
            def ev_body(e, ci=ci, p=p):
                tvals = t_v[pl.ds(ci * E + (e // 16) * 16, 16)]
                t = lax.gather(
                    tvals, jnp.broadcast_to(e % 16, (16, 1)),
                    lax.GatherDimensionNumbers(
                        offset_dims=(), collapsed_slice_dims=(0,),
                        start_index_map=(0,)),
                    (1,), mode=lax.GatherScatterMode.PROMISE_IN_BOUNDS)
                for h in range(HC):
                    hs = pl.ds(h * 16, 16)
                    v01 = rows_v[p, e * M + 0, hs] + rows_v[p, e * M + 1, hs]
                    v23 = rows_v[p, e * M + 2, hs] + rows_v[p, e * M + 3, hs]
                    out_v[p, e, hs] = v01 + v23 + base[h] + t * wch[h]
            out_pend[p] = pltpu.async_copy(
                out_v.at[p], out_hbm.at[pl.ds(b * S + ci * E, E)], osem[p])

        for p in range(2):
            if out_pend[p] is not None:
                out_pend[p].wait()
        return 0

    lax.fori_loop(0, ROWS_PER_W, row_body, 0)


@jax.jit
def _run(dyn_idx, static_idx, time, data_tbl, static_tbl, time_w, time_b):
    mesh = plsc.VectorSubcoreMesh(core_axis_name="c", subcore_axis_name="s")
    kfn = functools.partial(
        pl.kernel,
        mesh=mesh,
        out_type=jax.ShapeDtypeStruct((B * S, H), jnp.float32),
        scratch_types=[
            pltpu.VMEM((S * M,), jnp.int32),
            pltpu.VMEM((2, E * M, H), jnp.float32),
            pltpu.VMEM((2, E, H), jnp.float32),
            pltpu.VMEM((208,), jnp.float32),
            pltpu.VMEM((N_STATIC,), jnp.int32),
            pltpu.VMEM((N_STATIC, H), jnp.float32),
            pltpu.VMEM((2, H), jnp.float32),
            pltpu.SemaphoreType.DMA,
            pltpu.SemaphoreType.DMA,
            pltpu.SemaphoreType.DMA,
            pltpu.SemaphoreType.DMA,
            pltpu.SemaphoreType.DMA,
        ],
    )(_sc_body)
    return kfn(dyn_idx, static_idx, time, data_tbl, static_tbl, time_w, time_b)


def kernel(dynamic_indices, static_indices, time, data_emb_table,
           static_emb_table, time_w, time_b):
    out = _run(dynamic_indices.reshape(-1), static_indices, time.reshape(-1),
               data_emb_table, static_emb_table, time_w, time_b)
    return out.reshape(B, S, H)


# speedup vs baseline: 7.7010x; 1.0616x over previous
import functools

import jax
import jax.numpy as jnp
from jax import lax
from jax.experimental import pallas as pl
from jax.experimental.pallas import tpu as pltpu
from jax.experimental.pallas import tpu_sc as plsc

B, S, M, H = 1024, 200, 4, 128
N_STATIC = 8
NC, NS = 2, 16
NW = NC * NS
ROWS_PER_W = B // NW
E = 40
G = 20
NG = E // G
CHUNKS = S // E
HC = H // 16


def _sc_body(dyn_hbm, sidx_hbm, t_hbm, tbl_hbm, stbl_hbm, tw_hbm, tb_hbm,
             out_hbm, ridx_v, ridx2_v, rows_v, out_v, t_v, sidx_v, srows_v,
             wtb_v, gsem0, gsem1, ssem, osem0, osem1):
    wid = lax.axis_index("s") * NC + lax.axis_index("c")
    gsem = (gsem0, gsem1)
    osem = (osem0, osem1)

    pltpu.sync_copy(tw_hbm.at[0], wtb_v.at[0])
    pltpu.sync_copy(tb_hbm, wtb_v.at[1])
    wch = [wtb_v[0, pl.ds(h * 16, 16)] for h in range(HC)]
    tbch = [wtb_v[1, pl.ds(h * 16, 16)] for h in range(HC)]

    def start_gathers(ci, p):
        cps = []
        for m in range(M):
            cps.append(pltpu.async_copy(
                tbl_hbm.at[ridx_v.at[pl.ds(m * 208 + ci * E, E)]],
                rows_v.at[p, m], gsem[p]))
        return cps

    lanes = lax.iota(jnp.int32, 16)

    def row_body(r, _):
        b = wid * ROWS_PER_W + r
        pltpu.sync_copy(sidx_hbm.at[b], sidx_v)
        scp = pltpu.async_copy(stbl_hbm.at[sidx_v], srows_v, ssem)
        pltpu.sync_copy(dyn_hbm.at[b], ridx2_v.at[pl.ds(0, S)])
        for g in range(13):
            rid = lanes + (g * 16)
            for m in range(M):
                ridx_v[pl.ds(m * 208 + g * 16, 16)] = plsc.load_gather(
                    ridx2_v, [rid, jnp.full((16,), m, jnp.int32)])
        g_pend = {0: start_gathers(0, 0)}
        pltpu.sync_copy(t_hbm.at[pl.ds(b * S, S)], t_v.at[pl.ds(0, S)])
        scp.wait()

        base = []
        for h in range(HC):
            hs = pl.ds(h * 16, 16)
            acc = srows_v[0, hs]
            for j in range(1, N_STATIC):
                acc = acc + srows_v[j, hs]
            base.append(acc * (1.0 / N_STATIC) + tbch[h])

        out_pend = [None, None]
        for ci in range(CHUNKS):
            p = ci % 2
            if ci + 1 < CHUNKS:
                g_pend[ci + 1] = start_gathers(ci + 1, 1 - p)
            for cp in g_pend.pop(ci):
                cp.wait()
            if out_pend[p] is not None:
                out_pend[p].wait()
                out_pend[p] = None

            @plsc.parallel_loop(0, E, unroll=2)
            def ev_body(e, ci=ci, p=p):
                tvals = t_v[pl.ds(ci * E + (e // 16) * 16, 16)]
                t = lax.gather(
                    tvals, jnp.broadcast_to(e % 16, (16, 1)),
                    lax.GatherDimensionNumbers(
                        offset_dims=(), collapsed_slice_dims=(0,),
                        start_index_map=(0,)),
                    (1,), mode=lax.GatherScatterMode.PROMISE_IN_BOUNDS)
                for h in range(HC):
                    hs = pl.ds(h * 16, 16)
                    v01 = rows_v[p, 0, e, hs] + rows_v[p, 1, e, hs]
                    v23 = rows_v[p, 2, e, hs] + rows_v[p, 3, e, hs]
                    out_v[p, e, hs] = v01 + v23 + base[h] + t * wch[h]
            out_pend[p] = pltpu.async_copy(
                out_v.at[p], out_hbm.at[pl.ds(b * S + ci * E, E)], osem[p])

        for p in range(2):
            if out_pend[p] is not None:
                out_pend[p].wait()
        return 0

    lax.fori_loop(0, ROWS_PER_W, row_body, 0)


@jax.jit
def _run(dyn_idx, static_idx, time, data_tbl, static_tbl, time_w, time_b):
    mesh = plsc.VectorSubcoreMesh(core_axis_name="c", subcore_axis_name="s")
    kfn = functools.partial(
        pl.kernel,
        mesh=mesh,
        out_type=jax.ShapeDtypeStruct((B * S, H), jnp.float32),
        compiler_params=pltpu.CompilerParams(needs_layout_passes=False),
        scratch_types=[
            pltpu.VMEM((M * 208,), jnp.int32),
            pltpu.VMEM((208, M), jnp.int32),
            pltpu.VMEM((2, M, E, H), jnp.float32),
            pltpu.VMEM((2, E, H), jnp.float32),
            pltpu.VMEM((208,), jnp.float32),
            pltpu.VMEM((N_STATIC,), jnp.int32),
            pltpu.VMEM((N_STATIC, H), jnp.float32),
            pltpu.VMEM((2, H), jnp.float32),
            pltpu.SemaphoreType.DMA,
            pltpu.SemaphoreType.DMA,
            pltpu.SemaphoreType.DMA,
            pltpu.SemaphoreType.DMA,
            pltpu.SemaphoreType.DMA,
        ],
    )(_sc_body)
    return kfn(dyn_idx, static_idx, time, data_tbl, static_tbl, time_w, time_b)


def kernel(dynamic_indices, static_indices, time, data_emb_table,
           static_emb_table, time_w, time_b):
    out = _run(dynamic_indices, static_indices, time.reshape(-1),
               data_emb_table, static_emb_table, time_w, time_b)
    return out.reshape(B, S, H)


# speedup vs baseline: 7.9111x; 1.0273x over previous
import functools

import jax
import jax.numpy as jnp
from jax import lax
from jax.experimental import pallas as pl
from jax.experimental.pallas import tpu as pltpu
from jax.experimental.pallas import tpu_sc as plsc

B, S, M, H = 1024, 200, 4, 128
N_STATIC = 8
NC, NS = 2, 16
NW = NC * NS
ROWS_PER_W = B // NW
E = 40
CHUNKS = S // E
HC = H // 16
NBUF = 4


def _sc_body(dyn_hbm, sidx_hbm, t_hbm, tbl_hbm, stbl_hbm, tw_hbm, tb_hbm,
             out_hbm, ridx_v, ridx2_v, out_v, t_v, sidx_v, srows_v,
             wtb_v, gsem0, gsem1, gsem2, gsem3, osem0, osem1, osem2, osem3,
             ssem):
    wid = lax.axis_index("s") * NC + lax.axis_index("c")
    gsem = (gsem0, gsem1, gsem2, gsem3)
    osem = (osem0, osem1, osem2, osem3)

    pltpu.sync_copy(tw_hbm.at[0], wtb_v.at[0])
    pltpu.sync_copy(tb_hbm, wtb_v.at[1])
    wch = [wtb_v[0, pl.ds(h * 16, 16)] for h in range(HC)]
    tbch = [wtb_v[1, pl.ds(h * 16, 16)] for h in range(HC)]

    lanes = lax.iota(jnp.int32, 16)

    def row_body(r, _):
        b = wid * ROWS_PER_W + r
        pltpu.sync_copy(sidx_hbm.at[b], sidx_v)
        scp = pltpu.async_copy(stbl_hbm.at[sidx_v], srows_v, ssem)
        pltpu.sync_copy(dyn_hbm.at[b], ridx2_v.at[pl.ds(0, S)])
        for g in range(13):
            rid = lanes + (g * 16)
            for m in range(M):
                ridx_v[pl.ds(m * 208 + g * 16, 16)] = plsc.load_gather(
                    ridx2_v, [rid, jnp.full((16,), m, jnp.int32)])
        pltpu.sync_copy(t_hbm.at[pl.ds(b * S, S)], t_v.at[pl.ds(0, S)])
        scp.wait()

        base = []
        for h in range(HC):
            hs = pl.ds(h * 16, 16)
            acc = srows_v[0, hs]
            for j in range(1, N_STATIC):
                acc = acc + srows_v[j, hs]
            base.append(acc * (1.0 / N_STATIC) + tbch[h])

        g_pend = {}
        out_pend = {}
        for ci in range(CHUNKS):
            p = ci % NBUF
            if p in out_pend:
                out_pend.pop(p).wait()
            @plsc.parallel_loop(0, E, unroll=2)
            def pre_body(e, ci=ci, p=p):
                tvals = t_v[pl.ds(ci * E + (e // 16) * 16, 16)]
                t = lax.gather(
                    tvals, jnp.broadcast_to(e % 16, (16, 1)),
                    lax.GatherDimensionNumbers(
                        offset_dims=(), collapsed_slice_dims=(0,),
                        start_index_map=(0,)),
                    (1,), mode=lax.GatherScatterMode.PROMISE_IN_BOUNDS)
                for h in range(HC):
                    out_v[p, e, pl.ds(h * 16, 16)] = base[h] + t * wch[h]
            g_pend[ci] = [
                pltpu.async_copy(
                    tbl_hbm.at[ridx_v.at[pl.ds(m * 208 + ci * E, E)]],
                    out_v.at[p], gsem[p], add=True)
                for m in range(M)]
            if ci - 1 in g_pend:
                for cp in g_pend.pop(ci - 1):
                    cp.wait()
                q = (ci - 1) % NBUF
                out_pend[q] = pltpu.async_copy(
                    out_v.at[q], out_hbm.at[pl.ds(b * S + (ci - 1) * E, E)],
                    osem[q])

        for cp in g_pend.pop(CHUNKS - 1):
            cp.wait()
        q = (CHUNKS - 1) % NBUF
        out_pend[q] = pltpu.async_copy(
            out_v.at[q], out_hbm.at[pl.ds(b * S + (CHUNKS - 1) * E, E)],
            osem[q])
        for p in sorted(out_pend):
            out_pend[p].wait()
        return 0

    lax.fori_loop(0, ROWS_PER_W, row_body, 0)


@jax.jit
def _run(dyn_idx, static_idx, time, data_tbl, static_tbl, time_w, time_b):
    mesh = plsc.VectorSubcoreMesh(core_axis_name="c", subcore_axis_name="s")
    kfn = functools.partial(
        pl.kernel,
        mesh=mesh,
        out_type=jax.ShapeDtypeStruct((B * S, H), jnp.float32),
        compiler_params=pltpu.CompilerParams(needs_layout_passes=False),
        scratch_types=[
            pltpu.VMEM((M * 208,), jnp.int32),
            pltpu.VMEM((208, M), jnp.int32),
            pltpu.VMEM((NBUF, E, H), jnp.float32),
            pltpu.VMEM((208,), jnp.float32),
            pltpu.VMEM((N_STATIC,), jnp.int32),
            pltpu.VMEM((N_STATIC, H), jnp.float32),
            pltpu.VMEM((2, H), jnp.float32),
            pltpu.SemaphoreType.DMA,
            pltpu.SemaphoreType.DMA,
            pltpu.SemaphoreType.DMA,
            pltpu.SemaphoreType.DMA,
            pltpu.SemaphoreType.DMA,
            pltpu.SemaphoreType.DMA,
            pltpu.SemaphoreType.DMA,
            pltpu.SemaphoreType.DMA,
            pltpu.SemaphoreType.DMA,
        ],
    )(_sc_body)
    return kfn(dyn_idx, static_idx, time, data_tbl, static_tbl, time_w, time_b)


def kernel(dynamic_indices, static_indices, time, data_emb_table,
           static_emb_table, time_w, time_b):
    out = _run(dynamic_indices, static_indices, time.reshape(-1),
               data_emb_table, static_emb_table, time_w, time_b)
    return out.reshape(B, S, H)


# speedup vs baseline: 10.6183x; 1.3422x over previous
import functools

import jax
import jax.numpy as jnp
from jax import lax
from jax.experimental import pallas as pl
from jax.experimental.pallas import tpu as pltpu
from jax.experimental.pallas import tpu_sc as plsc

B, S, M, H = 1024, 200, 4, 128
N_STATIC = 8
NC, NS = 2, 16
NW = NC * NS
ROWS_PER_W = B // NW
E = 40
CHUNKS = S // E
HC = H // 16
SP = 208


def _sc_body(dyn_hbm, sidx_hbm, t_hbm, tbl_hbm, stbl_hbm, tw_hbm, tb_hbm,
             out_hbm, ridx_v, ridx2_v, out_v, t_v, sidxa_v, sidxb_v,
             srows_v, wtb_v, gsem0, gsem1, gsem2, gsem3, gsem4,
             osem0, osem1, osem2, osem3, osem4, ssem, tsem, stsem):
    wid = lax.axis_index("s") * NC + lax.axis_index("c")
    b0 = wid * ROWS_PER_W
    gsem = (gsem0, gsem1, gsem2, gsem3, gsem4)
    osem = (osem0, osem1, osem2, osem3, osem4)

    pltpu.sync_copy(tw_hbm.at[0], wtb_v.at[0])
    pltpu.sync_copy(tb_hbm, wtb_v.at[1])
    wch = [wtb_v[0, pl.ds(h * 16, 16)] for h in range(HC)]
    tbch = [wtb_v[1, pl.ds(h * 16, 16)] for h in range(HC)]

    lanes = lax.iota(jnp.int32, 16)

    tcps = [pltpu.async_copy(
        t_hbm.at[pl.ds((b0 + r) * S, S)], t_v.at[pl.ds(r * SP, S)], tsem)
        for r in range(ROWS_PER_W)]

    def stage_static(b, p):
        sidx = sidxa_v if p == 0 else sidxb_v
        pltpu.sync_copy(sidx_hbm.at[b], sidx)
        pltpu.async_copy(stbl_hbm.at[sidx], srows_v.at[p], ssem)

    def wait_static(p):
        pltpu.make_async_copy(
            stbl_hbm.at[pl.ds(0, N_STATIC)], srows_v.at[p], ssem).wait()

    def transpose_row(p):
        for g in range(13):
            rid = lanes + g * 16
            for m in range(M):
                ridx_v[pl.ds(p * M * SP + m * SP + g * 16, 16)] = \
                    plsc.load_gather(
                        ridx2_v, [rid, jnp.full((16,), m, jnp.int32)])

    stage_static(b0, 0)
    pltpu.sync_copy(dyn_hbm.at[b0], ridx2_v.at[pl.ds(0, S)])
    transpose_row(0)
    for cp in tcps:
        cp.wait()

    def wait_gathers(ci):
        for _m in range(M):
            pltpu.make_async_copy(
                tbl_hbm.at[pl.ds(0, E)], out_v.at[ci], gsem[ci]).wait()

    def wait_out(ci):
        pltpu.make_async_copy(
            out_v.at[ci], out_hbm.at[pl.ds(0, E)], osem[ci]).wait()

    def when(pred):
        if pred is True:
            return lambda fn: fn()
        return pl.when(pred)

    def do_row(r, not_first, not_last, p):
        b = b0 + r
        wait_static(p)
        base = []
        for h in range(HC):
            hs = pl.ds(h * 16, 16)
            acc = srows_v[p, 0, hs]
            for j in range(1, N_STATIC):
                acc = acc + srows_v[p, j, hs]
            base.append(acc * (1.0 / N_STATIC) + tbch[h])

        @when(not_last)
        def _():
            pltpu.async_copy(
                dyn_hbm.at[b + 1], ridx2_v.at[pl.ds(0, S)], stsem)
            stage_static(b + 1, 1 - p)

        g_pend = {}
        for ci in range(CHUNKS):
            @when(not_first)
            def _(ci=ci):
                wait_out(ci)
            @plsc.parallel_loop(0, E, unroll=2)
            def pre_body(e, ci=ci):
                tvals = t_v[pl.ds(r * SP + ci * E + (e // 16) * 16, 16)]
                t = lax.gather(
                    tvals, jnp.broadcast_to(e % 16, (16, 1)),
                    lax.GatherDimensionNumbers(
                        offset_dims=(), collapsed_slice_dims=(0,),
                        start_index_map=(0,)),
                    (1,), mode=lax.GatherScatterMode.PROMISE_IN_BOUNDS)
                for h in range(HC):
                    out_v[ci, e, pl.ds(h * 16, 16)] = base[h] + t * wch[h]
            g_pend[ci] = [
                pltpu.async_copy(
                    tbl_hbm.at[ridx_v.at[pl.ds(p * M * SP + m * SP + ci * E, E)]],
                    out_v.at[ci], gsem[ci], add=True)
                for m in range(M)]
            if ci > 0:
                for cp in g_pend.pop(ci - 1):
                    cp.wait()
                pltpu.async_copy(
                    out_v.at[ci - 1],
                    out_hbm.at[pl.ds(b * S + (ci - 1) * E, E)], osem[ci - 1])
            else:
                @when(not_first)
                def _():
                    wait_gathers(CHUNKS - 1)
                    pltpu.async_copy(
                        out_v.at[CHUNKS - 1],
                        out_hbm.at[pl.ds((b - 1) * S + (CHUNKS - 1) * E, E)],
                        osem[CHUNKS - 1])

        @when(not_last)
        def _():
            pltpu.make_async_copy(
                dyn_hbm.at[b], ridx2_v.at[pl.ds(0, S)], stsem).wait()
            transpose_row(1 - p)

    def pair_body(i, _):
        r0 = 2 * i
        do_row(r0, r0 > 0, True, 0)
        do_row(r0 + 1, True, r0 + 1 < ROWS_PER_W - 1, 1)
        return 0

    lax.fori_loop(0, ROWS_PER_W // 2, pair_body, 0)

    blast = b0 + ROWS_PER_W - 1
    wait_gathers(CHUNKS - 1)
    last = pltpu.async_copy(
        out_v.at[CHUNKS - 1],
        out_hbm.at[pl.ds(blast * S + (CHUNKS - 1) * E, E)],
        osem[CHUNKS - 1])
    for ci in range(CHUNKS - 1):
        wait_out(ci)
    last.wait()


@jax.jit
def _run(dyn_idx, static_idx, time, data_tbl, static_tbl, time_w, time_b):
    mesh = plsc.VectorSubcoreMesh(core_axis_name="c", subcore_axis_name="s")
    kfn = functools.partial(
        pl.kernel,
        mesh=mesh,
        out_type=jax.ShapeDtypeStruct((B * S, H), jnp.float32),
        compiler_params=pltpu.CompilerParams(needs_layout_passes=False),
        scratch_types=[
            pltpu.VMEM((2 * M * SP,), jnp.int32),
            pltpu.VMEM((SP, M), jnp.int32),
            pltpu.VMEM((CHUNKS, E, H), jnp.float32),
            pltpu.VMEM((ROWS_PER_W * SP,), jnp.float32),
            pltpu.VMEM((N_STATIC,), jnp.int32),
            pltpu.VMEM((N_STATIC,), jnp.int32),
            pltpu.VMEM((2, N_STATIC, H), jnp.float32),
            pltpu.VMEM((2, H), jnp.float32),
            pltpu.SemaphoreType.DMA,
            pltpu.SemaphoreType.DMA,
            pltpu.SemaphoreType.DMA,
            pltpu.SemaphoreType.DMA,
            pltpu.SemaphoreType.DMA,
            pltpu.SemaphoreType.DMA,
            pltpu.SemaphoreType.DMA,
            pltpu.SemaphoreType.DMA,
            pltpu.SemaphoreType.DMA,
            pltpu.SemaphoreType.DMA,
            pltpu.SemaphoreType.DMA,
            pltpu.SemaphoreType.DMA,
            pltpu.SemaphoreType.DMA,
        ],
    )(_sc_body)
    return kfn(dyn_idx, static_idx, time, data_tbl, static_tbl, time_w, time_b)


def kernel(dynamic_indices, static_indices, time, data_emb_table,
           static_emb_table, time_w, time_b):
    out = _run(dynamic_indices, static_indices, time.reshape(-1),
               data_emb_table, static_emb_table, time_w, time_b)
    return out.reshape(B, S, H)


# speedup vs baseline: 11.1401x; 1.0491x over previous
import functools

import jax
import jax.numpy as jnp
from jax import lax
from jax.experimental import pallas as pl
from jax.experimental.pallas import tpu as pltpu
from jax.experimental.pallas import tpu_sc as plsc

B, S, M, H = 1024, 200, 4, 128
N_STATIC = 8
NC, NS = 2, 16
NW = NC * NS
ROWS_PER_W = B // NW
E = 40
CHUNKS = S // E
HC = H // 16
SP = 208


def _sc_body(dyn_hbm, sidx_hbm, t_hbm, tbl_hbm, stbl_hbm, tw_hbm, tb_hbm,
             out_hbm, ridx_v, ridx2_v, out_v, t_v, sidxa_v, sidxb_v,
             srows_v, wtb_v, gsem0, gsem1, osem0, osem1, ssem, tsem, stsem):
    wid = lax.axis_index("s") * NC + lax.axis_index("c")
    b0 = wid * ROWS_PER_W
    gsem = (gsem0, gsem1)
    osem = (osem0, osem1)

    pltpu.sync_copy(tw_hbm.at[0], wtb_v.at[0])
    pltpu.sync_copy(tb_hbm, wtb_v.at[1])
    wch = [wtb_v[0, pl.ds(h * 16, 16)] for h in range(HC)]
    tbch = [wtb_v[1, pl.ds(h * 16, 16)] for h in range(HC)]

    lanes = lax.iota(jnp.int32, 16)

    tcps = [pltpu.async_copy(
        t_hbm.at[pl.ds((b0 + r) * S, S)], t_v.at[pl.ds(r * SP, S)], tsem)
        for r in range(ROWS_PER_W)]

    def stage_static(b, p):
        sidx = sidxa_v if p == 0 else sidxb_v
        pltpu.sync_copy(sidx_hbm.at[b], sidx)
        pltpu.async_copy(stbl_hbm.at[sidx], srows_v.at[p], ssem)

    def wait_static(p):
        pltpu.make_async_copy(
            stbl_hbm.at[pl.ds(0, N_STATIC)], srows_v.at[p], ssem).wait()

    def transpose_row(p):
        for g in range(13):
            rid = lanes + g * 16
            for m in range(M):
                ridx_v[pl.ds(p * M * SP + m * SP + g * 16, 16)] = \
                    plsc.load_gather(
                        ridx2_v, [rid, jnp.full((16,), m, jnp.int32)])

    stage_static(b0, 0)
    pltpu.sync_copy(dyn_hbm.at[b0], ridx2_v.at[pl.ds(0, S)])
    transpose_row(0)
    for cp in tcps:
        cp.wait()

    def wait_gathers(p):
        for _ in range(CHUNKS * M):
            pltpu.make_async_copy(
                tbl_hbm.at[pl.ds(0, E)], out_v.at[0], gsem[p]).wait()

    def wait_outs(p):
        for _ in range(CHUNKS):
            pltpu.make_async_copy(
                out_v.at[0], out_hbm.at[pl.ds(0, E)], osem[p]).wait()

    def when(pred):
        if pred is True:
            return lambda fn: fn()
        return pl.when(pred)

    def do_row(r, not_first, not_first2, not_last, p):
        b = b0 + r
        wait_static(p)
        base = []
        for h in range(HC):
            hs = pl.ds(h * 16, 16)
            acc = srows_v[p, 0, hs]
            for j in range(1, N_STATIC):
                acc = acc + srows_v[p, j, hs]
            base.append(acc * (1.0 / N_STATIC) + tbch[h])

        @when(not_last)
        def _():
            pltpu.async_copy(
                dyn_hbm.at[b + 1], ridx2_v.at[pl.ds(0, S)], stsem)
            stage_static(b + 1, 1 - p)

        @when(not_first2)
        def _():
            wait_outs(p)

        for ci in range(CHUNKS):
            @plsc.parallel_loop(0, E, unroll=2)
            def pre_body(e, ci=ci):
                tvals = t_v[pl.ds(r * SP + ci * E + (e // 16) * 16, 16)]
                t = lax.gather(
                    tvals, jnp.broadcast_to(e % 16, (16, 1)),
                    lax.GatherDimensionNumbers(
                        offset_dims=(), collapsed_slice_dims=(0,),
                        start_index_map=(0,)),
                    (1,), mode=lax.GatherScatterMode.PROMISE_IN_BOUNDS)
                for h in range(HC):
                    out_v[p * CHUNKS + ci, e, pl.ds(h * 16, 16)] = \
                        base[h] + t * wch[h]
        for ci in range(CHUNKS):
            for m in range(M):
                pltpu.async_copy(
                    tbl_hbm.at[ridx_v.at[pl.ds(p * M * SP + m * SP + ci * E, E)]],
                    out_v.at[p * CHUNKS + ci], gsem[p], add=True)
        @when(not_first)
        def _():
            wait_gathers(1 - p)
            for ci in range(CHUNKS):
                pltpu.async_copy(
                    out_v.at[(1 - p) * CHUNKS + ci],
                    out_hbm.at[pl.ds((b - 1) * S + ci * E, E)], osem[1 - p])

        @when(not_last)
        def _():
            pltpu.make_async_copy(
                dyn_hbm.at[b], ridx2_v.at[pl.ds(0, S)], stsem).wait()
            transpose_row(1 - p)

    def pair_body(i, _):
        r0 = 2 * i
        do_row(r0, i > 0, i > 0, True, 0)
        do_row(r0 + 1, True, i > 0, i < ROWS_PER_W // 2 - 1, 1)
        return 0

    lax.fori_loop(0, ROWS_PER_W // 2, pair_body, 0)

    blast = b0 + ROWS_PER_W - 1
    wait_gathers(1)
    for ci in range(CHUNKS):
        pltpu.async_copy(
            out_v.at[CHUNKS + ci],
            out_hbm.at[pl.ds(blast * S + ci * E, E)], osem[1])
    wait_outs(0)
    wait_outs(1)


@jax.jit
def _run(dyn_idx, static_idx, time, data_tbl, static_tbl, time_w, time_b):
    mesh = plsc.VectorSubcoreMesh(core_axis_name="c", subcore_axis_name="s")
    kfn = functools.partial(
        pl.kernel,
        mesh=mesh,
        out_type=jax.ShapeDtypeStruct((B * S, H), jnp.float32),
        compiler_params=pltpu.CompilerParams(needs_layout_passes=False),
        scratch_types=[
            pltpu.VMEM((2 * M * SP,), jnp.int32),
            pltpu.VMEM((SP, M), jnp.int32),
            pltpu.VMEM((2 * CHUNKS, E, H), jnp.float32),
            pltpu.VMEM((ROWS_PER_W * SP,), jnp.float32),
            pltpu.VMEM((N_STATIC,), jnp.int32),
            pltpu.VMEM((N_STATIC,), jnp.int32),
            pltpu.VMEM((2, N_STATIC, H), jnp.float32),
            pltpu.VMEM((2, H), jnp.float32),
            pltpu.SemaphoreType.DMA,
            pltpu.SemaphoreType.DMA,
            pltpu.SemaphoreType.DMA,
            pltpu.SemaphoreType.DMA,
            pltpu.SemaphoreType.DMA,
            pltpu.SemaphoreType.DMA,
            pltpu.SemaphoreType.DMA,
        ],
    )(_sc_body)
    return kfn(dyn_idx, static_idx, time, data_tbl, static_tbl, time_w, time_b)


def kernel(dynamic_indices, static_indices, time, data_emb_table,
           static_emb_table, time_w, time_b):
    out = _run(dynamic_indices, static_indices, time.reshape(-1),
               data_emb_table, static_emb_table, time_w, time_b)
    return out.reshape(B, S, H)
